# Initial kernel scaffold; baseline (speedup 1.0000x reference)
import jax
import jax.numpy as jnp
from jax.experimental import pallas as pl


def kernel(q, mu, edge_index, edge_weight, W1, b1, W2, b2, Wf, bf):
    raise NotImplementedError("write your pallas kernel here")


# speedup vs baseline: 6.0441x; 6.0441x over previous
import functools
import math

import jax
import jax.numpy as jnp
from jax.experimental import pallas as pl
from jax.experimental.pallas import tpu as pltpu
from jax.experimental.pallas import tpu_sc as plsc

_CUTOFF = 5.0
_NSLICE = 4
_CH = 64
_NSUB = 16
_NCORE = 2


def _node_prep_kernel(q_ref, mu_ref, w1t_ref, b1_ref, w2t_ref, b2_ref,
                      nodes_ref, base_ref):
    x1 = jnp.dot(q_ref[...], w1t_ref[...], preferred_element_type=jnp.float32)
    x1 = x1 + b1_ref[...]
    x1 = x1 * jax.nn.sigmoid(x1)
    x = jnp.dot(x1, w2t_ref[...], preferred_element_type=jnp.float32)
    x = x + b2_ref[...]
    nb = q_ref.shape[1]
    sl = nb // _NSLICE
    for s in range(_NSLICE):
        nodes_ref[s, :, 0 * sl:1 * sl] = x[:, 0 * nb + sl * s:0 * nb + sl * s + sl]
        nodes_ref[s, :, 1 * sl:2 * sl] = x[:, 1 * nb + sl * s:1 * nb + sl * s + sl]
        nodes_ref[s, :, 2 * sl:3 * sl] = x[:, 2 * nb + sl * s:2 * nb + sl * s + sl]
        base_ref[s, :, 0:sl] = q_ref[:, sl * s:sl * s + sl]
        for d in range(3):
            m = mu_ref[:, nb * d + sl * s:nb * d + sl * s + sl]
            nodes_ref[s, :, (3 + d) * sl:(4 + d) * sl] = m
            base_ref[s, :, sl + sl * d:2 * sl + sl * d] = m
        nodes_ref[s, :, 6 * sl:8 * sl] = jnp.zeros_like(q_ref[:, 0:2 * sl])


def _edge_prep_kernel(ew_ref, wft_ref, bf_ref, fsa_ref, fsb_ref, *, nr, nb):
    ew = ew_ref[...]
    be = ew.shape[0]
    d2 = jnp.sum(ew * ew, axis=1, keepdims=True)
    dist = jnp.sqrt(d2)
    inv = 1.0 / dist
    dirs = ew * inv
    n = jax.lax.broadcasted_iota(jnp.int32, (be, nr), 1).astype(jnp.float32) + 1.0
    basis = jnp.sin(dist * n * (math.pi / _CUTOFF)) * inv
    cut = jnp.where(dist < _CUTOFF,
                    0.5 * (jnp.cos(dist * (math.pi / _CUTOFF)) + 1.0), 0.0)
    filt = jnp.dot(basis, wft_ref[...], preferred_element_type=jnp.float32)
    filt = (filt + bf_ref[...]) * cut
    sl = nb // _NSLICE
    for s in range(_NSLICE):
        fsa_ref[s, :, 0:sl] = filt[:, sl * s:sl * s + sl]
        fR = filt[:, nb + sl * s:nb + sl * s + sl]
        for d in range(3):
            fsa_ref[s, :, sl + sl * d:2 * sl + sl * d] = fR * dirs[:, d:d + 1]
        fsb_ref[s, :, 0:sl] = filt[:, 2 * nb + sl * s:2 * nb + sl * s + sl]


def _sc_kernel(nodes_hbm, fsa_hbm, fsb_hbm, base_hbm, i3_hbm, j3_hbm,
               out_hbm, accum, fab, fbb, gb, ub, ib, jb, jb2,
               *, n_nodes, n_edges, sl):
    core = jax.lax.axis_index("c")
    sub = jax.lax.axis_index("s")
    rpt = (n_nodes // _NSUB) & ~7
    tail = n_nodes - rpt * _NSUB
    n_chunks = n_edges // _CH
    cpt = -(-n_chunks // _NSUB)
    for s_local in range(_NSLICE // _NCORE):
        sidx = core * (_NSLICE // _NCORE) + s_local
        pltpu.sync_copy(base_hbm.at[sidx, pl.ds(sub * rpt, rpt)],
                        accum.at[pl.ds(sub * rpt, rpt)])
        if tail:
            @pl.when(sub == _NSUB - 1)
            def _init_tail():
                pltpu.sync_copy(base_hbm.at[sidx, pl.ds(rpt * _NSUB, tail)],
                                accum.at[pl.ds(rpt * _NSUB, tail)])
        plsc.subcore_barrier()
        joff = sidx * n_nodes

        @pl.loop(0, cpt)
        def _chunk(k):
            chunk = k * _NSUB + sub

            @pl.when(chunk < n_chunks)
            def _do_chunk():
                e0 = chunk * _CH
                pltpu.sync_copy(i3_hbm.at[chunk], ib)
                pltpu.sync_copy(j3_hbm.at[chunk], jb)
                pltpu.sync_copy(fsa_hbm.at[sidx, pl.ds(e0, _CH)], fab)
                pltpu.sync_copy(fsb_hbm.at[sidx, pl.ds(e0, _CH)], fbb)

                @pl.loop(0, _CH, step=16)
                def _adj(c):
                    jb2[0, pl.ds(c, 16)] = jb[0, pl.ds(c, 16)] + joff

                pltpu.sync_copy(nodes_hbm.at[jb2.at[0]], gb)

                @pl.loop(0, _CH)
                def _edge(e):
                    for c in range(0, sl, 16):
                        xq = gb[e, pl.ds(c, 16)]
                        xR = gb[e, pl.ds(sl + c, 16)]
                        xM = gb[e, pl.ds(2 * sl + c, 16)]
                        ub[e, pl.ds(c, 16)] = fab[e, pl.ds(c, 16)] * xq
                        dM = fbb[e, pl.ds(c, 16)] * xM
                        for d in range(3):
                            fRd = fab[e, pl.ds(sl + sl * d + c, 16)]
                            mv = gb[e, pl.ds(3 * sl + sl * d + c, 16)]
                            ub[e, pl.ds(sl + sl * d + c, 16)] = fRd * xR + dM * mv

                pltpu.sync_copy(ub, accum.at[ib.at[0]], add=True)

        plsc.subcore_barrier()
        pltpu.sync_copy(accum.at[pl.ds(sub * rpt, rpt)],
                        out_hbm.at[sidx, pl.ds(sub * rpt, rpt)])
        if tail:
            @pl.when(sub == _NSUB - 1)
            def _out_tail():
                pltpu.sync_copy(accum.at[pl.ds(rpt * _NSUB, tail)],
                                out_hbm.at[sidx, pl.ds(rpt * _NSUB, tail)])
        plsc.subcore_barrier()


def kernel(q, mu, edge_index, edge_weight, W1, b1, W2, b2, Wf, bf):
    n, nb = q.shape
    e = edge_index.shape[1]
    nr = Wf.shape[1]
    sl = nb // _NSLICE
    bn = 400
    be = 2000

    mu2 = mu.reshape(n, 3 * nb)
    nodes, base = pl.pallas_call(
        _node_prep_kernel,
        grid=(n // bn,),
        in_specs=[
            pl.BlockSpec((bn, nb), lambda i: (i, 0)),
            pl.BlockSpec((bn, 3 * nb), lambda i: (i, 0)),
            pl.BlockSpec((nb, nb), lambda i: (0, 0)),
            pl.BlockSpec((1, nb), lambda i: (0, 0)),
            pl.BlockSpec((nb, 3 * nb), lambda i: (0, 0)),
            pl.BlockSpec((1, 3 * nb), lambda i: (0, 0)),
        ],
        out_specs=[
            pl.BlockSpec((_NSLICE, bn, 8 * sl), lambda i: (0, i, 0)),
            pl.BlockSpec((_NSLICE, bn, nb), lambda i: (0, i, 0)),
        ],
        out_shape=[
            jax.ShapeDtypeStruct((_NSLICE, n, 8 * sl), jnp.float32),
            jax.ShapeDtypeStruct((_NSLICE, n, nb), jnp.float32),
        ],
    )(q, mu2, W1.T, b1.reshape(1, nb), W2.T, b2.reshape(1, 3 * nb))

    fsa, fsb = pl.pallas_call(
        functools.partial(_edge_prep_kernel, nr=nr, nb=nb),
        grid=(e // be,),
        in_specs=[
            pl.BlockSpec((be, 3), lambda i: (i, 0)),
            pl.BlockSpec((nr, 3 * nb), lambda i: (0, 0)),
            pl.BlockSpec((1, 3 * nb), lambda i: (0, 0)),
        ],
        out_specs=[
            pl.BlockSpec((_NSLICE, be, 4 * sl), lambda i: (0, i, 0)),
            pl.BlockSpec((_NSLICE, be, sl), lambda i: (0, i, 0)),
        ],
        out_shape=[
            jax.ShapeDtypeStruct((_NSLICE, e, 4 * sl), jnp.float32),
            jax.ShapeDtypeStruct((_NSLICE, e, sl), jnp.float32),
        ],
    )(edge_weight, Wf.T, bf.reshape(1, 3 * nb))

    nodes_flat = nodes.reshape(_NSLICE * n, 8 * sl)
    i3 = edge_index[0].reshape(e // _CH, 1, _CH)
    j3 = edge_index[1].reshape(e // _CH, 1, _CH)

    sc = pl.kernel(
        functools.partial(_sc_kernel, n_nodes=n, n_edges=e, sl=sl),
        out_type=jax.ShapeDtypeStruct((_NSLICE, n, nb), jnp.float32),
        mesh=plsc.VectorSubcoreMesh(core_axis_name="c", subcore_axis_name="s"),
        scratch_types=[
            pltpu.VMEM_SHARED((n, nb), jnp.float32),
            pltpu.VMEM((_CH, 4 * sl), jnp.float32),
            pltpu.VMEM((_CH, sl), jnp.float32),
            pltpu.VMEM((_CH, 8 * sl), jnp.float32),
            pltpu.VMEM((_CH, nb), jnp.float32),
            pltpu.VMEM((1, _CH), jnp.int32),
            pltpu.VMEM((1, _CH), jnp.int32),
            pltpu.VMEM((1, _CH), jnp.int32),
        ],
    )
    out = sc(nodes_flat, fsa, fsb, base, i3, j3)

    q_out = jnp.concatenate([out[s, :, 0:sl] for s in range(_NSLICE)], axis=1)
    mu_out = jnp.stack(
        [jnp.concatenate([out[s, :, sl + sl * d:2 * sl + sl * d]
                          for s in range(_NSLICE)], axis=1)
         for d in range(3)], axis=1)
    return (q_out, mu_out)


# speedup vs baseline: 6.6433x; 1.0991x over previous
import functools
import math

import jax
import jax.numpy as jnp
from jax.experimental import pallas as pl
from jax.experimental.pallas import tpu as pltpu
from jax.experimental.pallas import tpu_sc as plsc

_CUTOFF = 5.0
_NSLICE = 4
_CH = 64
_NSUB = 16
_NCORE = 2


def _node_prep_kernel(q_ref, mu_ref, w1t_ref, b1_ref, w2t_ref, b2_ref,
                      nodes_ref, base_ref):
    x1 = jnp.dot(q_ref[...], w1t_ref[...], preferred_element_type=jnp.float32)
    x1 = x1 + b1_ref[...]
    x1 = x1 * jax.nn.sigmoid(x1)
    x = jnp.dot(x1, w2t_ref[...], preferred_element_type=jnp.float32)
    x = x + b2_ref[...]
    nb = q_ref.shape[1]
    sl = nb // _NSLICE
    for s in range(_NSLICE):
        nodes_ref[s, :, 0 * sl:1 * sl] = x[:, 0 * nb + sl * s:0 * nb + sl * s + sl]
        nodes_ref[s, :, 1 * sl:2 * sl] = x[:, 1 * nb + sl * s:1 * nb + sl * s + sl]
        nodes_ref[s, :, 2 * sl:3 * sl] = x[:, 2 * nb + sl * s:2 * nb + sl * s + sl]
        base_ref[s, :, 0:sl] = q_ref[:, sl * s:sl * s + sl]
        for d in range(3):
            m = mu_ref[:, nb * d + sl * s:nb * d + sl * s + sl]
            nodes_ref[s, :, (3 + d) * sl:(4 + d) * sl] = m
            base_ref[s, :, sl + sl * d:2 * sl + sl * d] = m
        nodes_ref[s, :, 6 * sl:8 * sl] = jnp.zeros_like(q_ref[:, 0:2 * sl])


def _edge_prep_kernel(ew_ref, wft_ref, bf_ref, fs_ref, *, nr, nb):
    ew = ew_ref[...]
    be = ew.shape[0]
    d2 = jnp.sum(ew * ew, axis=1, keepdims=True)
    dist = jnp.sqrt(d2)
    inv = 1.0 / dist
    dirs = ew * inv
    n = jax.lax.broadcasted_iota(jnp.int32, (be, nr), 1).astype(jnp.float32) + 1.0
    basis = jnp.sin(dist * n * (math.pi / _CUTOFF)) * inv
    cut = jnp.where(dist < _CUTOFF,
                    0.5 * (jnp.cos(dist * (math.pi / _CUTOFF)) + 1.0), 0.0)
    filt = jnp.dot(basis, wft_ref[...], preferred_element_type=jnp.float32)
    filt = (filt + bf_ref[...]) * cut
    sl = nb // _NSLICE
    for s in range(_NSLICE):
        fs_ref[s, :, 0:sl] = filt[:, sl * s:sl * s + sl]
        fR = filt[:, nb + sl * s:nb + sl * s + sl]
        for d in range(3):
            fs_ref[s, :, sl + sl * d:2 * sl + sl * d] = fR * dirs[:, d:d + 1]
        fs_ref[s, :, 4 * sl:5 * sl] = filt[:, 2 * nb + sl * s:2 * nb + sl * s + sl]


def _sc_kernel(nodes_hbm, fs_hbm, base_hbm, ij_hbm,
               out_hbm, accum, fb, gb, ub, ijb, sb, jb2,
               *, n_nodes, n_edges, sl):
    core = jax.lax.axis_index("c")
    sub = jax.lax.axis_index("s")
    rpt = (n_nodes // _NSUB) & ~7
    tail = n_nodes - rpt * _NSUB
    n_chunks = n_edges // _CH
    cpt = -(-n_chunks // _NSUB)
    for s_local in range(_NSLICE // _NCORE):
        sidx = core * (_NSLICE // _NCORE) + s_local
        pltpu.sync_copy(base_hbm.at[sidx, pl.ds(sub * rpt, rpt)],
                        accum.at[pl.ds(sub * rpt, rpt)])
        if tail:
            @pl.when(sub == _NSUB - 1)
            def _init_tail():
                pltpu.sync_copy(base_hbm.at[sidx, pl.ds(rpt * _NSUB, tail)],
                                accum.at[pl.ds(rpt * _NSUB, tail)])
        plsc.subcore_barrier()
        joff = sidx * n_nodes

        @pl.loop(0, cpt)
        def _chunk(k):
            chunk = k * _NSUB + sub

            @pl.when(chunk < n_chunks)
            def _do_chunk():
                e0 = chunk * _CH
                pltpu.sync_copy(ij_hbm.at[chunk], ijb)
                pltpu.sync_copy(fs_hbm.at[sidx, pl.ds(e0, _CH)], fb)

                @pl.loop(0, _CH, step=16)
                def _adj(c):
                    sb[0, pl.ds(c, 16)] = ijb[0, pl.ds(c, 16)]
                    jb2[0, pl.ds(c, 16)] = ijb[1, pl.ds(c, 16)] + joff

                pltpu.sync_copy(nodes_hbm.at[jb2.at[0]], gb)

                @pl.loop(0, _CH)
                def _edge(e):
                    for c in range(0, sl, 16):
                        xq = gb[e, pl.ds(c, 16)]
                        xR = gb[e, pl.ds(sl + c, 16)]
                        xM = gb[e, pl.ds(2 * sl + c, 16)]
                        ub[e, pl.ds(c, 16)] = fb[e, pl.ds(c, 16)] * xq
                        dM = fb[e, pl.ds(4 * sl + c, 16)] * xM
                        for d in range(3):
                            fRd = fb[e, pl.ds(sl + sl * d + c, 16)]
                            mv = gb[e, pl.ds(3 * sl + sl * d + c, 16)]
                            ub[e, pl.ds(sl + sl * d + c, 16)] = fRd * xR + dM * mv

                pltpu.sync_copy(ub, accum.at[sb.at[0]], add=True)

        plsc.subcore_barrier()
        pltpu.sync_copy(accum.at[pl.ds(sub * rpt, rpt)],
                        out_hbm.at[sidx, pl.ds(sub * rpt, rpt)])
        if tail:
            @pl.when(sub == _NSUB - 1)
            def _out_tail():
                pltpu.sync_copy(accum.at[pl.ds(rpt * _NSUB, tail)],
                                out_hbm.at[sidx, pl.ds(rpt * _NSUB, tail)])
        plsc.subcore_barrier()


def kernel(q, mu, edge_index, edge_weight, W1, b1, W2, b2, Wf, bf):
    n, nb = q.shape
    e = edge_index.shape[1]
    nr = Wf.shape[1]
    sl = nb // _NSLICE
    bn = 400
    be = 2000

    mu2 = mu.reshape(n, 3 * nb)
    nodes, base = pl.pallas_call(
        _node_prep_kernel,
        grid=(n // bn,),
        in_specs=[
            pl.BlockSpec((bn, nb), lambda i: (i, 0)),
            pl.BlockSpec((bn, 3 * nb), lambda i: (i, 0)),
            pl.BlockSpec((nb, nb), lambda i: (0, 0)),
            pl.BlockSpec((1, nb), lambda i: (0, 0)),
            pl.BlockSpec((nb, 3 * nb), lambda i: (0, 0)),
            pl.BlockSpec((1, 3 * nb), lambda i: (0, 0)),
        ],
        out_specs=[
            pl.BlockSpec((_NSLICE, bn, 8 * sl), lambda i: (0, i, 0)),
            pl.BlockSpec((_NSLICE, bn, nb), lambda i: (0, i, 0)),
        ],
        out_shape=[
            jax.ShapeDtypeStruct((_NSLICE, n, 8 * sl), jnp.float32),
            jax.ShapeDtypeStruct((_NSLICE, n, nb), jnp.float32),
        ],
    )(q, mu2, W1.T, b1.reshape(1, nb), W2.T, b2.reshape(1, 3 * nb))

    fs = pl.pallas_call(
        functools.partial(_edge_prep_kernel, nr=nr, nb=nb),
        grid=(e // be,),
        in_specs=[
            pl.BlockSpec((be, 3), lambda i: (i, 0)),
            pl.BlockSpec((nr, 3 * nb), lambda i: (0, 0)),
            pl.BlockSpec((1, 3 * nb), lambda i: (0, 0)),
        ],
        out_specs=pl.BlockSpec((_NSLICE, be, 5 * sl), lambda i: (0, i, 0)),
        out_shape=jax.ShapeDtypeStruct((_NSLICE, e, 5 * sl), jnp.float32),
    )(edge_weight, Wf.T, bf.reshape(1, 3 * nb))

    nodes_flat = nodes.reshape(_NSLICE * n, 8 * sl)
    ij = jnp.stack([edge_index[0].reshape(e // _CH, _CH),
                    edge_index[1].reshape(e // _CH, _CH)], axis=1)

    sc = pl.kernel(
        functools.partial(_sc_kernel, n_nodes=n, n_edges=e, sl=sl),
        out_type=jax.ShapeDtypeStruct((_NSLICE, n, nb), jnp.float32),
        mesh=plsc.VectorSubcoreMesh(core_axis_name="c", subcore_axis_name="s"),
        scratch_types=[
            pltpu.VMEM_SHARED((n, nb), jnp.float32),
            pltpu.VMEM((_CH, 5 * sl), jnp.float32),
            pltpu.VMEM((_CH, 8 * sl), jnp.float32),
            pltpu.VMEM((_CH, nb), jnp.float32),
            pltpu.VMEM((2, _CH), jnp.int32),
            pltpu.VMEM((1, _CH), jnp.int32),
            pltpu.VMEM((1, _CH), jnp.int32),
        ],
    )
    out = sc(nodes_flat, fs, base, ij)

    q_out = jnp.concatenate([out[s, :, 0:sl] for s in range(_NSLICE)], axis=1)
    mu_out = jnp.stack(
        [jnp.concatenate([out[s, :, sl + sl * d:2 * sl + sl * d]
                          for s in range(_NSLICE)], axis=1)
         for d in range(3)], axis=1)
    return (q_out, mu_out)
